# baseline (speedup 1.0000x reference)
import functools

import jax
import jax.numpy as jnp
import numpy as np
from jax import lax
from jax.experimental import pallas as pl
from jax.experimental.pallas import tpu as pltpu
from jax.experimental.pallas import tpu_sc as plsc

_N = 1048576
_G = 128
_NC = 2
_NS = 16
_NW = _NC * _NS
_B = _N // _NW
_L = 16
_NCH = 4
_CH = _B // _NCH
_EPS = np.float32(1e-10)
_SCALE = np.float32(_G - 1)
_GM2 = np.int32(_G - 2)


def _tec_body(obs_hbm, grid_hbm, out_hbm, obs_v, grid_v, out_v,
              gsem, osem, *isems):
    wid = lax.axis_index("s") * _NC + lax.axis_index("c")
    base = wid * _B
    cg = pltpu.async_copy(grid_hbm, grid_v, gsem)
    ins = [
        pltpu.async_copy(
            obs_hbm.at[pl.ds(2 * (base + c * _CH), 2 * _CH)],
            obs_v.at[pl.ds(2 * c * _CH, 2 * _CH)],
            isems[c])
        for c in range(_NCH)
    ]
    cg.wait()
    for c in ins:
        c.wait()
    _chunk(obs_v, grid_v, out_v, 0)
    pltpu.sync_copy(out_v, out_hbm.at[0, pl.ds(base, _B)])


def _chunk(obs_v, grid_v, out_v, start):
    @plsc.parallel_loop(start, start + _B, _L, unroll=8)
    def _loop(o):
        addr = o + ((o >> 7) << 7)
        u = obs_v[pl.ds(addr, _L)]
        v = obs_v[pl.ds(addr + 128, _L)]

        fu = u * _SCALE
        fv = v * _SCALE
        i0u = jnp.minimum(fu.astype(jnp.int32), _GM2)
        i0v = jnp.minimum(fv.astype(jnp.int32), _GM2)
        du = fu - i0u.astype(jnp.float32)
        dv = fv - i0v.astype(jnp.float32)

        a00 = (i0u << 7) + i0v
        a10 = a00 + _G
        g00 = plsc.load_gather(grid_v, [a00])
        g01 = plsc.load_gather(grid_v, [a00 + 1])
        g10 = plsc.load_gather(grid_v, [a10])
        g11 = plsc.load_gather(grid_v, [a10 + 1])

        top = g00 + (g01 - g00) * dv
        bot = g10 + (g11 - g10) * dv
        out_v[pl.ds(o, _L)] = top + (bot - top) * du


def kernel(obs, pdf_grid):
    mesh = plsc.VectorSubcoreMesh(core_axis_name="c", subcore_axis_name="s")
    run = functools.partial(
        pl.kernel,
        mesh=mesh,
        out_type=jax.ShapeDtypeStruct((1, _N), jnp.float32),
        compiler_params=pltpu.CompilerParams(
            needs_layout_passes=False, use_tc_tiling_on_sc=False),
        scratch_types=[
            pltpu.VMEM((2 * _B,), jnp.float32),
            pltpu.VMEM((_G * _G,), jnp.float32),
            pltpu.VMEM((_B,), jnp.float32),
        ] + [pltpu.SemaphoreType.DMA] * (2 + _NCH),
    )(_tec_body)
    obs_flat = obs.reshape(_N // _G, _G, 2).transpose(0, 2, 1).reshape(-1)
    return run(obs_flat, pdf_grid.reshape(-1)).reshape(_N, 1)


# speedup vs baseline: 1.0795x; 1.0795x over previous
import functools

import jax
import jax.numpy as jnp
import numpy as np
from jax import lax
from jax.experimental import pallas as pl
from jax.experimental.pallas import tpu as pltpu
from jax.experimental.pallas import tpu_sc as plsc

_N = 1048576
_G = 128
_NC = 2
_NS = 16
_NW = _NC * _NS
_B = _N // _NW
_L = 16
_NCH = 4
_CH = _B // _NCH
_EPS = np.float32(1e-10)
_SCALE = np.float32(_G - 1)
_GM2 = np.int32(_G - 2)


def _tec_body(obs_hbm, grid_hbm, out_hbm, obs_v, grid_v, out_v,
              gsem, osem, *isems):
    wid = lax.axis_index("s") * _NC + lax.axis_index("c")
    base = wid * _B
    cg = pltpu.async_copy(grid_hbm, grid_v, gsem)
    ins = [
        pltpu.async_copy(
            obs_hbm.at[pl.ds(2 * (base + c * _CH), 2 * _CH)],
            obs_v.at[pl.ds(2 * c * _CH, 2 * _CH)],
            isems[c])
        for c in range(_NCH)
    ]
    cg.wait()
    for c in ins:
        c.wait()
    _chunk(obs_v, grid_v, out_v, 0)
    pltpu.sync_copy(out_v, out_hbm.at[0, pl.ds(base, _B)])


def _chunk(obs_v, grid_v, out_v, start):
    @plsc.parallel_loop(start, start + _B, _L, unroll=8)
    def _loop(o):
        addr = o + ((o >> 7) << 7)
        u = obs_v[pl.ds(addr, _L)]
        v = obs_v[pl.ds(addr + 128, _L)]

        fu = jnp.maximum(u, _EPS) * _SCALE
        fv = jnp.maximum(v, _EPS) * _SCALE
        i0u = jnp.minimum(fu.astype(jnp.int32), _GM2)
        i0v = jnp.minimum(fv.astype(jnp.int32), _GM2)
        du = fu - i0u.astype(jnp.float32)
        dv = fv - i0v.astype(jnp.float32)

        a00 = (i0u << 7) + i0v
        a10 = a00 + _G
        g00 = plsc.load_gather(grid_v, [a00])
        g01 = plsc.load_gather(grid_v, [a00 + 1])
        g10 = plsc.load_gather(grid_v, [a10])
        g11 = plsc.load_gather(grid_v, [a10 + 1])

        top = g00 + (g01 - g00) * dv
        bot = g10 + (g11 - g10) * dv
        res = top + (bot - top) * du
        out_v[pl.ds(o, _L)] = jnp.maximum(res, np.float32(0.0))


def kernel(obs, pdf_grid):
    mesh = plsc.VectorSubcoreMesh(core_axis_name="c", subcore_axis_name="s")
    run = functools.partial(
        pl.kernel,
        mesh=mesh,
        out_type=jax.ShapeDtypeStruct((1, _N), jnp.float32),
        compiler_params=pltpu.CompilerParams(
            needs_layout_passes=False, use_tc_tiling_on_sc=False),
        scratch_types=[
            pltpu.VMEM((2 * _B,), jnp.float32),
            pltpu.VMEM((_G * _G,), jnp.float32),
            pltpu.VMEM((_B,), jnp.float32),
        ] + [pltpu.SemaphoreType.DMA] * (2 + _NCH),
    )(_tec_body)
    obs_flat = obs.reshape(_N // _G, _G, 2).transpose(0, 2, 1).reshape(-1)
    return run(obs_flat, pdf_grid.reshape(-1)).reshape(_N, 1)


# speedup vs baseline: 1.1297x; 1.0465x over previous
import functools

import jax
import jax.numpy as jnp
import numpy as np
from jax import lax
from jax.experimental import pallas as pl
from jax.experimental.pallas import tpu as pltpu
from jax.experimental.pallas import tpu_sc as plsc

_N = 1048576
_G = 128
_NC = 2
_NS = 16
_NW = _NC * _NS
_B = _N // _NW
_L = 16
_NCH = 4
_CH = _B // _NCH
_EPS = np.float32(1e-10)
_SCALE = np.float32(_G - 1)
_GM2U = np.uint32(_G - 2)


def _tec_body(obs_hbm, grid_hbm, out_hbm, obs_v, grid_v, out_v,
              gsem, osem, *isems):
    wid = lax.axis_index("s") * _NC + lax.axis_index("c")
    base = wid * _B
    cg = pltpu.async_copy(grid_hbm, grid_v, gsem)
    ins = [
        pltpu.async_copy(
            obs_hbm.at[pl.ds(2 * (base + c * _CH), 2 * _CH)],
            obs_v.at[pl.ds(2 * c * _CH, 2 * _CH)],
            isems[c])
        for c in range(_NCH)
    ]
    cg.wait()
    for c in ins:
        c.wait()
    _chunk(obs_v, grid_v, out_v, 0)
    pltpu.sync_copy(out_v, out_hbm.at[0, pl.ds(base, _B)])


def _chunk(obs_v, grid_v, out_v, start):
    @plsc.parallel_loop(start, start + _B, _L, unroll=8)
    def _loop(o):
        addr = o + ((o >> 7) << 7)
        u = obs_v[pl.ds(addr, _L)]
        v = obs_v[pl.ds(addr + 128, _L)]

        fu = jnp.maximum(u, _EPS) * _SCALE
        fv = jnp.maximum(v, _EPS) * _SCALE
        i0u = plsc.bitcast(
            jnp.minimum(plsc.bitcast(fu.astype(jnp.int32), jnp.uint32), _GM2U),
            jnp.int32)
        i0v = plsc.bitcast(
            jnp.minimum(plsc.bitcast(fv.astype(jnp.int32), jnp.uint32), _GM2U),
            jnp.int32)
        du = fu - i0u.astype(jnp.float32)
        dv = fv - i0v.astype(jnp.float32)

        a00 = (i0u << 7) + i0v
        a10 = a00 + _G
        g00 = plsc.load_gather(grid_v, [a00])
        g01 = plsc.load_gather(grid_v, [a00 + 1])
        g10 = plsc.load_gather(grid_v, [a10])
        g11 = plsc.load_gather(grid_v, [a10 + 1])

        top = g00 + (g01 - g00) * dv
        bot = g10 + (g11 - g10) * dv
        res = top + (bot - top) * du
        out_v[pl.ds(o, _L)] = jnp.maximum(res, np.float32(0.0))


def kernel(obs, pdf_grid):
    mesh = plsc.VectorSubcoreMesh(core_axis_name="c", subcore_axis_name="s")
    run = functools.partial(
        pl.kernel,
        mesh=mesh,
        out_type=jax.ShapeDtypeStruct((1, _N), jnp.float32),
        compiler_params=pltpu.CompilerParams(
            needs_layout_passes=False, use_tc_tiling_on_sc=False),
        scratch_types=[
            pltpu.VMEM((2 * _B,), jnp.float32),
            pltpu.VMEM((_G * _G,), jnp.float32),
            pltpu.VMEM((_B,), jnp.float32),
        ] + [pltpu.SemaphoreType.DMA] * (2 + _NCH),
    )(_tec_body)
    obs_flat = obs.reshape(_N // _G, _G, 2).transpose(0, 2, 1).reshape(-1)
    return run(obs_flat, pdf_grid.reshape(-1)).reshape(_N, 1)
